# Initial kernel scaffold; baseline (speedup 1.0000x reference)
import jax
import jax.numpy as jnp
from jax.experimental import pallas as pl


def kernel(input_ids, pos_s, pos_e, word_table, lin_W, lin_b, ln_g, ln_b, rel_table, fus_W, fus_b):
    raise NotImplementedError("write your pallas kernel here")


# speedup vs baseline: 1.0731x; 1.0731x over previous
import functools

import jax
import jax.numpy as jnp
from jax import lax
from jax.experimental import pallas as pl
from jax.experimental.pallas import tpu as pltpu
from jax.experimental.pallas import tpu_sc as plsc

NC, NS, L = 2, 16, 16
NW = NC * NS
HIDDEN = 64
SEQ = 200
SEQP = 208
NPOS = 401
MAXLEN = 200
TBL = NPOS * HIDDEN
EPS = 1e-12

_MESH = plsc.VectorSubcoreMesh(core_axis_name="c", subcore_axis_name="s")
_SC_PARAMS = pltpu.CompilerParams(needs_layout_passes=False)


def _word_gather(idx_pad, word_table):
    n = idx_pad.shape[0]
    bpw = n // NW

    @functools.partial(
        pl.kernel, mesh=_MESH,
        out_type=jax.ShapeDtypeStruct((n, HIDDEN), jnp.float32),
        compiler_params=_SC_PARAMS,
        scratch_types=[
            pltpu.VMEM((bpw,), jnp.int32),
            pltpu.VMEM((bpw, HIDDEN), jnp.float32),
        ],
    )
    def k(table_hbm, idx_hbm, out_hbm, idx_v, rows_v):
        wid = lax.axis_index("s") * NC + lax.axis_index("c")
        base = wid * bpw
        pltpu.sync_copy(idx_hbm.at[pl.ds(base, bpw)], idx_v)

        def row_body(r, c):
            rv = plsc.load_gather(idx_v, [jnp.broadcast_to(r, (L,))])
            rid = jnp.max(rv)
            pltpu.sync_copy(table_hbm.at[rid], rows_v.at[r])
            return c

        lax.fori_loop(0, bpw, row_body, 0)
        pltpu.sync_copy(rows_v, out_hbm.at[pl.ds(base, bpw)])

    return k(word_table, idx_pad)


def _tc_x(rows, lin_W, lin_b2, g2, b2):
    def body(r_ref, w_ref, lb_ref, g_ref, bb_ref, x_ref):
        x = lax.dot_general(r_ref[...], w_ref[...],
                            (((1,), (1,)), ((), ())),
                            preferred_element_type=jnp.float32)
        x = x + lb_ref[...]
        mu = jnp.mean(x, axis=1, keepdims=True)
        xc = x - mu
        var = jnp.mean(xc * xc, axis=1, keepdims=True)
        x_ref[...] = xc * lax.rsqrt(var + EPS) * g_ref[...] + bb_ref[...]

    return pl.pallas_call(
        body, out_shape=jax.ShapeDtypeStruct(rows.shape, jnp.float32),
    )(rows, lin_W, lin_b2, g2, b2)


def _tc_tables(rel_table, fus_W, fus_b2):
    def body(rel_ref, w_ref, b_ref, out_ref):
        rel = rel_ref[...]
        w = w_ref[...]
        for kk in range(4):
            wk = w[:, kk * HIDDEN:(kk + 1) * HIDDEN]
            pk = lax.dot_general(rel, wk, (((1,), (1,)), ((), ())),
                                 preferred_element_type=jnp.float32)
            if kk == 0:
                pk = pk + b_ref[...]
            out_ref[pl.ds(kk * NPOS, NPOS), :] = pk

    return pl.pallas_call(
        body, out_shape=jax.ShapeDtypeStruct((4 * NPOS, HIDDEN), jnp.float32),
    )(rel_table, fus_W, fus_b2)


def _relpos(p_flat, ps_pad, pe_pad):
    items_per = (4 * SEQ) // NW
    nchunks = SEQP // L

    @functools.partial(
        pl.kernel, mesh=_MESH,
        out_type=jax.ShapeDtypeStruct((4, SEQ, SEQ, HIDDEN), jnp.float32),
        compiler_params=_SC_PARAMS,
        scratch_types=[
            pltpu.VMEM((4 * TBL,), jnp.float32),
            pltpu.VMEM((4 * SEQP,), jnp.int32),
            pltpu.VMEM((4 * SEQP,), jnp.int32),
            pltpu.VMEM((SEQP, HIDDEN), jnp.float32),
        ],
    )
    def k(p_hbm, ps_hbm, pe_hbm, out_hbm, p_v, ps_v, pe_v, buf):
        wid = lax.axis_index("s") * NC + lax.axis_index("c")
        pltpu.sync_copy(p_hbm, p_v)
        pltpu.sync_copy(ps_hbm, ps_v)
        pltpu.sync_copy(pe_hbm, pe_v)
        lanes = lax.broadcasted_iota(jnp.int32, (L,), 0)

        def item_body(tt, carry):
            t = wid * items_per + tt
            b = t // SEQ
            i = t - b * SEQ
            pb = b * SEQP
            ivec = jnp.broadcast_to(pb + i, (L,))
            s_i = plsc.load_gather(ps_v, [ivec])
            e_i = plsc.load_gather(pe_v, [ivec])

            def chunk_body(jc, c2):
                off = pb + jc * L
                s_j = ps_v[pl.ds(off, L)]
                e_j = pe_v[pl.ds(off, L)]
                base0 = (s_i - s_j) * HIDDEN + (MAXLEN * HIDDEN)
                base1 = (s_i - e_j) * HIDDEN + (MAXLEN * HIDDEN + TBL)
                base2 = (e_i - s_j) * HIDDEN + (MAXLEN * HIDDEN + 2 * TBL)
                base3 = (e_i - e_j) * HIDDEN + (MAXLEN * HIDDEN + 3 * TBL)
                jl = jc * L + lanes
                for d in range(HIDDEN):
                    g0 = plsc.load_gather(p_v, [base0 + d])
                    g1 = plsc.load_gather(p_v, [base1 + d])
                    g2 = plsc.load_gather(p_v, [base2 + d])
                    g3 = plsc.load_gather(p_v, [base3 + d])
                    v = jnp.maximum((g0 + g1) + (g2 + g3), 0.0)
                    dv = jnp.full((L,), d, jnp.int32)
                    plsc.store_scatter(buf, [jl, dv], v)
                return c2

            lax.fori_loop(0, nchunks, chunk_body, 0)
            pltpu.sync_copy(buf.at[pl.ds(0, SEQ)], out_hbm.at[b, i])
            return carry

        lax.fori_loop(0, items_per, item_body, 0)

    return k(p_flat, ps_pad, pe_pad)


def kernel(input_ids, pos_s, pos_e, word_table, lin_W, lin_b, ln_g, ln_b,
           rel_table, fus_W, fus_b):
    n_tok = input_ids.size
    n_pad = NW * 32
    ids_pad = jnp.pad(input_ids.reshape(-1), (0, n_pad - n_tok))
    rows = _word_gather(ids_pad.astype(jnp.int32), word_table)
    x = _tc_x(rows, lin_W, lin_b.reshape(1, -1), ln_g.reshape(1, -1),
              ln_b.reshape(1, -1))
    x = x[:n_tok].reshape(input_ids.shape + (HIDDEN,))

    p_tab = _tc_tables(rel_table, fus_W, fus_b.reshape(1, -1))
    ps_pad = jnp.pad(pos_s, ((0, 0), (0, SEQP - SEQ))).reshape(-1)
    pe_pad = jnp.pad(pos_e, ((0, 0), (0, SEQP - SEQ))).reshape(-1)
    rel = _relpos(p_tab.reshape(-1), ps_pad.astype(jnp.int32),
                  pe_pad.astype(jnp.int32))
    return x, rel


# speedup vs baseline: 1.4138x; 1.3174x over previous
import functools

import jax
import jax.numpy as jnp
from jax import lax
from jax.experimental import pallas as pl
from jax.experimental.pallas import tpu as pltpu
from jax.experimental.pallas import tpu_sc as plsc

NC, NS, L = 2, 16, 16
NW = NC * NS
HIDDEN = 64
SEQ = 200
SEQP = 208
NPOS = 401
MAXLEN = 200
TBL = NPOS * HIDDEN
ROWS_A = 96
ROWS_B = SEQ - ROWS_A
EPS = 1e-12

_MESH = plsc.VectorSubcoreMesh(core_axis_name="c", subcore_axis_name="s")
_SC_PARAMS = pltpu.CompilerParams(needs_layout_passes=False)


def _word_gather(idx_pad, word_table):
    n = idx_pad.shape[0]
    bpw = n // NW

    @functools.partial(
        pl.kernel, mesh=_MESH,
        out_type=jax.ShapeDtypeStruct((n, HIDDEN), jnp.float32),
        compiler_params=_SC_PARAMS,
        scratch_types=[
            pltpu.VMEM((bpw,), jnp.int32),
            pltpu.VMEM((bpw, HIDDEN), jnp.float32),
        ],
    )
    def k(table_hbm, idx_hbm, out_hbm, idx_v, rows_v):
        wid = lax.axis_index("s") * NC + lax.axis_index("c")
        base = wid * bpw
        pltpu.sync_copy(idx_hbm.at[pl.ds(base, bpw)], idx_v)

        def row_body(r, c):
            rv = plsc.load_gather(idx_v, [jnp.broadcast_to(r, (L,))])
            rid = jnp.max(rv)
            pltpu.sync_copy(table_hbm.at[rid], rows_v.at[r])
            return c

        lax.fori_loop(0, bpw, row_body, 0)
        pltpu.sync_copy(rows_v, out_hbm.at[pl.ds(base, bpw)])

    return k(word_table, idx_pad)


def _tc_x(rows, lin_W, lin_b2, g2, b2):
    def body(r_ref, w_ref, lb_ref, g_ref, bb_ref, x_ref):
        x = lax.dot_general(r_ref[...], w_ref[...],
                            (((1,), (1,)), ((), ())),
                            preferred_element_type=jnp.float32)
        x = x + lb_ref[...]
        mu = jnp.mean(x, axis=1, keepdims=True)
        xc = x - mu
        var = jnp.mean(xc * xc, axis=1, keepdims=True)
        x_ref[...] = xc * lax.rsqrt(var + EPS) * g_ref[...] + bb_ref[...]

    return pl.pallas_call(
        body, out_shape=jax.ShapeDtypeStruct(rows.shape, jnp.float32),
    )(rows, lin_W, lin_b2, g2, b2)


def _tc_tables(rel_table, fus_W, fus_b2):
    def body(rel_ref, w_ref, b_ref, out_ref):
        rel = rel_ref[...]
        w = w_ref[...]
        for kk in range(4):
            wk = w[:, kk * HIDDEN:(kk + 1) * HIDDEN]
            pk = lax.dot_general(rel, wk, (((1,), (1,)), ((), ())),
                                 preferred_element_type=jnp.float32)
            if kk == 0:
                pk = pk + b_ref[...]
            out_ref[pl.ds(kk * NPOS, NPOS), :] = pk

    return pl.pallas_call(
        body, out_shape=jax.ShapeDtypeStruct((4 * NPOS, HIDDEN), jnp.float32),
    )(rel_table, fus_W, fus_b2)


def _relpos(p_flat, ps_pad, pe_pad):
    items_per = (4 * SEQ) // NW
    nchunks = SEQP // L

    @functools.partial(
        pl.kernel, mesh=_MESH,
        out_type=jax.ShapeDtypeStruct((4, SEQ, SEQ, HIDDEN), jnp.float32),
        compiler_params=_SC_PARAMS,
        scratch_types=[
            pltpu.VMEM((4 * TBL,), jnp.float32),
            pltpu.VMEM((4 * SEQP,), jnp.int32),
            pltpu.VMEM((4 * SEQP,), jnp.int32),
            pltpu.VMEM((ROWS_A, HIDDEN), jnp.float32),
            pltpu.VMEM((ROWS_B, HIDDEN), jnp.float32),
            pltpu.SemaphoreType.DMA,
            pltpu.SemaphoreType.DMA,
        ],
    )
    def k(p_hbm, ps_hbm, pe_hbm, out_hbm, p_v, ps_v, pe_v, buf_a, buf_b,
          sem_a, sem_b):
        wid = lax.axis_index("s") * NC + lax.axis_index("c")
        pltpu.sync_copy(p_hbm, p_v)
        pltpu.sync_copy(ps_hbm, ps_v)
        pltpu.sync_copy(pe_hbm, pe_v)
        lanes = lax.broadcasted_iota(jnp.int32, (L,), 0)

        def do_item(t, do_wait):
            b = t // SEQ
            i = t - b * SEQ
            pb = b * SEQP
            ivec = jnp.broadcast_to(pb + i, (L,))
            s_i = plsc.load_gather(ps_v, [ivec])
            e_i = plsc.load_gather(pe_v, [ivec])

            def half(bslot, sem, jc_lo, jc_hi, row_off, nrows):
                if do_wait:
                    pltpu.make_async_copy(
                        bslot, out_hbm.at[b, i, pl.ds(row_off, nrows)],
                        sem).wait()

                def chunk_body(jc, c2):
                    off = pb + jc * L
                    s_j = ps_v[pl.ds(off, L)]
                    e_j = pe_v[pl.ds(off, L)]
                    base0 = (s_i - s_j) * HIDDEN + (MAXLEN * HIDDEN)
                    base1 = (s_i - e_j) * HIDDEN + (MAXLEN * HIDDEN + TBL)
                    base2 = (e_i - s_j) * HIDDEN + (MAXLEN * HIDDEN + 2 * TBL)
                    base3 = (e_i - e_j) * HIDDEN + (MAXLEN * HIDDEN + 3 * TBL)
                    jl = jc * L + lanes - row_off
                    msk = jl < nrows

                    @plsc.parallel_loop(0, HIDDEN, unroll=4)
                    def d_body(d):
                        g0 = plsc.load_gather(p_v, [base0 + d])
                        g1 = plsc.load_gather(p_v, [base1 + d])
                        g2 = plsc.load_gather(p_v, [base2 + d])
                        g3 = plsc.load_gather(p_v, [base3 + d])
                        v = jnp.maximum((g0 + g1) + (g2 + g3), 0.0)
                        dv = jnp.broadcast_to(d, (L,))
                        plsc.store_scatter(bslot, [jl, dv], v, mask=msk)

                    return c2

                lax.fori_loop(jc_lo, jc_hi, chunk_body, 0)
                pltpu.async_copy(
                    bslot, out_hbm.at[b, i, pl.ds(row_off, nrows)], sem)

            half(buf_a, sem_a, 0, ROWS_A // L, 0, ROWS_A)
            half(buf_b, sem_b, ROWS_A // L, nchunks, ROWS_A, ROWS_B)

        t0 = wid * items_per

        def item_loop(tt, c):
            do_item(t0 + tt, True)
            return c

        do_item(t0, False)
        lax.fori_loop(1, items_per, item_loop, 0)
        pltpu.make_async_copy(buf_a, out_hbm.at[0, 0, pl.ds(0, ROWS_A)],
                              sem_a).wait()
        pltpu.make_async_copy(buf_b, out_hbm.at[0, 0, pl.ds(ROWS_A, ROWS_B)],
                              sem_b).wait()

    return k(p_flat, ps_pad, pe_pad)


def kernel(input_ids, pos_s, pos_e, word_table, lin_W, lin_b, ln_g, ln_b,
           rel_table, fus_W, fus_b):
    n_tok = input_ids.size
    n_pad = NW * 32
    ids_pad = jnp.pad(input_ids.reshape(-1), (0, n_pad - n_tok))
    rows = _word_gather(ids_pad.astype(jnp.int32), word_table)
    x = _tc_x(rows, lin_W, lin_b.reshape(1, -1), ln_g.reshape(1, -1),
              ln_b.reshape(1, -1))
    x = x[:n_tok].reshape(input_ids.shape + (HIDDEN,))

    p_tab = _tc_tables(rel_table, fus_W, fus_b.reshape(1, -1))
    ps_pad = jnp.pad(pos_s, ((0, 0), (0, SEQP - SEQ))).reshape(-1)
    pe_pad = jnp.pad(pos_e, ((0, 0), (0, SEQP - SEQ))).reshape(-1)
    rel = _relpos(p_tab.reshape(-1), ps_pad.astype(jnp.int32),
                  pe_pad.astype(jnp.int32))
    return x, rel


# speedup vs baseline: 3.1795x; 2.2490x over previous
import functools

import jax
import jax.numpy as jnp
from jax import lax
from jax.experimental import pallas as pl
from jax.experimental.pallas import tpu as pltpu
from jax.experimental.pallas import tpu_sc as plsc

NC, NS, L = 2, 16, 16
NW = NC * NS
HIDDEN = 64
SEQ = 200
SEQP = 208
NPOS = 401
MAXLEN = 200
TBL = NPOS * HIDDEN
ROWS_A = 96
ROWS_B = SEQ - ROWS_A
EPS = 1e-12

_MESH = plsc.VectorSubcoreMesh(core_axis_name="c", subcore_axis_name="s")
_SC_PARAMS = pltpu.CompilerParams(needs_layout_passes=False)


def _word_gather(idx_pad, word_table):
    n = idx_pad.shape[0]
    bpw = n // NW

    @functools.partial(
        pl.kernel, mesh=_MESH,
        out_type=jax.ShapeDtypeStruct((n, HIDDEN), jnp.float32),
        compiler_params=_SC_PARAMS,
        scratch_types=[
            pltpu.VMEM((bpw,), jnp.int32),
            pltpu.VMEM((bpw, HIDDEN), jnp.float32),
        ],
    )
    def k(table_hbm, idx_hbm, out_hbm, idx_v, rows_v):
        wid = lax.axis_index("s") * NC + lax.axis_index("c")
        base = wid * bpw
        pltpu.sync_copy(idx_hbm.at[pl.ds(base, bpw)], idx_v)

        def row_body(r, c):
            rv = plsc.load_gather(idx_v, [jnp.broadcast_to(r, (L,))])
            rid = jnp.max(rv)
            pltpu.sync_copy(table_hbm.at[rid], rows_v.at[r])
            return c

        lax.fori_loop(0, bpw, row_body, 0)
        pltpu.sync_copy(rows_v, out_hbm.at[pl.ds(base, bpw)])

    return k(word_table, idx_pad)


def _tc_x(rows, lin_W, lin_b2, g2, b2):
    def body(r_ref, w_ref, lb_ref, g_ref, bb_ref, x_ref):
        x = lax.dot_general(r_ref[...], w_ref[...],
                            (((1,), (1,)), ((), ())),
                            preferred_element_type=jnp.float32)
        x = x + lb_ref[...]
        mu = jnp.mean(x, axis=1, keepdims=True)
        xc = x - mu
        var = jnp.mean(xc * xc, axis=1, keepdims=True)
        x_ref[...] = xc * lax.rsqrt(var + EPS) * g_ref[...] + bb_ref[...]

    return pl.pallas_call(
        body, out_shape=jax.ShapeDtypeStruct(rows.shape, jnp.float32),
    )(rows, lin_W, lin_b2, g2, b2)


def _tc_tables(rel_table, fus_W, fus_b2):
    def body(rel_ref, w_ref, b_ref, out_ref):
        rel = rel_ref[...]
        w = w_ref[...]
        for kk in range(4):
            wk = w[:, kk * HIDDEN:(kk + 1) * HIDDEN]
            pk = lax.dot_general(rel, wk, (((1,), (1,)), ((), ())),
                                 preferred_element_type=jnp.float32)
            if kk == 0:
                pk = pk + b_ref[...]
            out_ref[pl.ds(kk * NPOS, NPOS), :] = pk

    return pl.pallas_call(
        body, out_shape=jax.ShapeDtypeStruct((4 * NPOS, HIDDEN), jnp.float32),
    )(rel_table, fus_W, fus_b2)


def _relpos(p_flat, ps_pad, pe_pad):
    items_per = (4 * SEQ) // NW
    nchunks = SEQP // L

    @functools.partial(
        pl.kernel, mesh=_MESH,
        out_type=jax.ShapeDtypeStruct((4, SEQ, SEQ, HIDDEN), jnp.float32),
        compiler_params=_SC_PARAMS,
        scratch_types=[
            pltpu.VMEM((4 * TBL,), jnp.float32),
            pltpu.VMEM((4 * SEQP,), jnp.int32),
            pltpu.VMEM((4 * SEQP,), jnp.int32),
            pltpu.VMEM((ROWS_A, HIDDEN), jnp.float32),
            pltpu.VMEM((ROWS_B, HIDDEN), jnp.float32),
            pltpu.SemaphoreType.DMA,
            pltpu.SemaphoreType.DMA,
        ],
    )
    def k(p_hbm, ps_hbm, pe_hbm, out_hbm, p_v, ps_v, pe_v, buf_a, buf_b,
          sem_a, sem_b):
        wid = lax.axis_index("s") * NC + lax.axis_index("c")
        pltpu.sync_copy(p_hbm, p_v)
        pltpu.sync_copy(ps_hbm, ps_v)
        pltpu.sync_copy(pe_hbm, pe_v)
        lanes = lax.broadcasted_iota(jnp.int32, (L,), 0)

        def do_item(t, do_wait):
            b = t // SEQ
            i = t - b * SEQ
            pb = b * SEQP
            ivec = jnp.broadcast_to(pb + i, (L,))
            s_i = plsc.load_gather(ps_v, [ivec])
            e_i = plsc.load_gather(pe_v, [ivec])

            def half(bslot, sem, jc_lo, jc_hi, row_off, nrows):
                if do_wait:
                    pltpu.make_async_copy(
                        bslot, out_hbm.at[b, i, pl.ds(row_off, nrows)],
                        sem).wait()

                def chunk_body(jc, c2):
                    off = pb + jc * L
                    s_j = ps_v[pl.ds(off, L)]
                    e_j = pe_v[pl.ds(off, L)]
                    base0 = (s_i - s_j) * HIDDEN + (MAXLEN * HIDDEN)
                    base1 = (s_i - e_j) * HIDDEN + (MAXLEN * HIDDEN + TBL)
                    base2 = (e_i - s_j) * HIDDEN + (MAXLEN * HIDDEN + 2 * TBL)
                    base3 = (e_i - e_j) * HIDDEN + (MAXLEN * HIDDEN + 3 * TBL)
                    jl = jc * L + lanes - row_off
                    msk = jl < nrows

                    @plsc.parallel_loop(0, HIDDEN, unroll=4)
                    def d_body(d):
                        c = (lanes + d) & 63
                        g0 = plsc.load_gather(p_v, [base0 + c])
                        g1 = plsc.load_gather(p_v, [base1 + c])
                        g2 = plsc.load_gather(p_v, [base2 + c])
                        g3 = plsc.load_gather(p_v, [base3 + c])
                        v = jnp.maximum((g0 + g1) + (g2 + g3), 0.0)
                        plsc.store_scatter(bslot, [jl, c], v, mask=msk)

                    return c2

                lax.fori_loop(jc_lo, jc_hi, chunk_body, 0)
                pltpu.async_copy(
                    bslot, out_hbm.at[b, i, pl.ds(row_off, nrows)], sem)

            half(buf_a, sem_a, 0, ROWS_A // L, 0, ROWS_A)
            half(buf_b, sem_b, ROWS_A // L, nchunks, ROWS_A, ROWS_B)

        t0 = wid * items_per

        def item_loop(tt, c):
            do_item(t0 + tt, True)
            return c

        do_item(t0, False)
        lax.fori_loop(1, items_per, item_loop, 0)
        pltpu.make_async_copy(buf_a, out_hbm.at[0, 0, pl.ds(0, ROWS_A)],
                              sem_a).wait()
        pltpu.make_async_copy(buf_b, out_hbm.at[0, 0, pl.ds(ROWS_A, ROWS_B)],
                              sem_b).wait()

    return k(p_flat, ps_pad, pe_pad)


def kernel(input_ids, pos_s, pos_e, word_table, lin_W, lin_b, ln_g, ln_b,
           rel_table, fus_W, fus_b):
    n_tok = input_ids.size
    n_pad = NW * 32
    ids_pad = jnp.pad(input_ids.reshape(-1), (0, n_pad - n_tok))
    rows = _word_gather(ids_pad.astype(jnp.int32), word_table)
    x = _tc_x(rows, lin_W, lin_b.reshape(1, -1), ln_g.reshape(1, -1),
              ln_b.reshape(1, -1))
    x = x[:n_tok].reshape(input_ids.shape + (HIDDEN,))

    p_tab = _tc_tables(rel_table, fus_W, fus_b.reshape(1, -1))
    ps_pad = jnp.pad(pos_s, ((0, 0), (0, SEQP - SEQ))).reshape(-1)
    pe_pad = jnp.pad(pos_e, ((0, 0), (0, SEQP - SEQ))).reshape(-1)
    rel = _relpos(p_tab.reshape(-1), ps_pad.astype(jnp.int32),
                  pe_pad.astype(jnp.int32))
    return x, rel
